# Initial kernel scaffold; baseline (speedup 1.0000x reference)
import jax
import jax.numpy as jnp
from jax.experimental import pallas as pl


def kernel(nfeats, coordinates, efeats, edge_index, node_graph_ids, We1, be1, We2, be2, Wx, bx, Wh1, bh1, Wh2, bh2, Wm, bm, Wd, bd, Wq, bq, Wo, bo):
    raise NotImplementedError("write your pallas kernel here")


# speedup vs baseline: 4.3430x; 4.3430x over previous
import functools

import jax
import jax.numpy as jnp
import numpy as np
from jax import lax
from jax.experimental import pallas as pl
from jax.experimental.pallas import tpu as pltpu
import jax.experimental.pallas.tpu_sc as plsc

_N = 10000
_E = 320000
_F = 128
_De = 16
_H = 128
_G = 100
_NCONV = 20

_TN = 2000
_TE = 2000
_GP = 104

_NW = 32
_EPW = _E // _NW
_CEG = 400
_NCHG = _EPW // _CEG
_CES = 200
_NCHS = _EPW // _CES
_NP = 10240
_NPS = _NP // 16

_f32 = jnp.float32
_u32 = jnp.uint32
_bf16 = jnp.bfloat16


def _silu(x):
    return x * jax.nn.sigmoid(x)


def _prep_body(h_ref, a_ref, b_ref, ps_ref, pd_ref):
    h = h_ref[...]
    ps_ref[...] = jnp.dot(h, a_ref[...], preferred_element_type=_f32)
    pd_ref[...] = jnp.dot(h, b_ref[...], preferred_element_type=_f32)


def _tc_prep(h, A, B):
    return pl.pallas_call(
        _prep_body,
        grid=(_N // _TN,),
        in_specs=[
            pl.BlockSpec((_TN, _F), lambda i: (i, 0)),
            pl.BlockSpec((_F, _F), lambda i: (0, 0)),
            pl.BlockSpec((_F, _F), lambda i: (0, 0)),
        ],
        out_specs=[
            pl.BlockSpec((_TN, _F), lambda i: (i, 0)),
            pl.BlockSpec((_TN, _F), lambda i: (i, 0)),
        ],
        out_shape=[jax.ShapeDtypeStruct((_N, _F), _f32)] * 2,
    )(h, A, B)


def _pack_table(P, xp):
    pw = lax.bitcast_convert_type(
        P.astype(_bf16).reshape(_N, _F // 2, 2), _u32)
    xw = lax.bitcast_convert_type(xp[:, 0:3], _u32)
    return jnp.concatenate(
        [pw, xw, jnp.zeros((_N, 61), _u32)], axis=1)


_SC_MESH = plsc.VectorSubcoreMesh(core_axis_name="c", subcore_axis_name="s")


@functools.partial(
    pl.kernel,
    out_type=[
        jax.ShapeDtypeStruct((_E, _F), _u32),
        jax.ShapeDtypeStruct((_E, _F), _u32),
    ],
    mesh=_SC_MESH,
    scratch_types=[
        pltpu.VMEM((_CEG,), jnp.int32),
        pltpu.VMEM((_CEG,), jnp.int32),
        pltpu.VMEM((_CEG, _F), _u32),
        pltpu.VMEM((_CEG, _F), _u32),
        pltpu.SemaphoreType.DMA,
        pltpu.SemaphoreType.DMA,
    ],
)
def _sc_gather(ts_hbm, td_hbm, src_hbm, dst_hbm,
               gs_hbm, gd_hbm,
               isv, idv, bps, bpd, s1, s2):
    cid = lax.axis_index("c")
    sid = lax.axis_index("s")
    wbase = (cid * 16 + sid) * _EPW

    def body(c, carry):
        base = pl.multiple_of(wbase + c * _CEG, 8)
        pltpu.sync_copy(src_hbm.at[pl.ds(base, _CEG)], isv)
        pltpu.sync_copy(dst_hbm.at[pl.ds(base, _CEG)], idv)
        c1 = pltpu.async_copy(ts_hbm.at[isv], bps, s1)
        c2 = pltpu.async_copy(td_hbm.at[idv], bpd, s2)
        c1.wait()
        c2.wait()
        pltpu.sync_copy(bps, gs_hbm.at[pl.ds(base, _CEG)])
        pltpu.sync_copy(bpd, gd_hbm.at[pl.ds(base, _CEG)])
        return carry

    lax.fori_loop(0, _NCHG, body, 0)


@functools.partial(
    pl.kernel,
    out_type=jax.ShapeDtypeStruct((2, _NP, _F), _f32),
    mesh=_SC_MESH,
    scratch_types=[
        pltpu.VMEM_SHARED((_NP, _F), _f32),
        pltpu.VMEM((_CES, _F), _f32),
        pltpu.VMEM((_CES,), jnp.int32),
    ],
)
def _sc_scatter(pay_hbm, dst_hbm, zero_hbm, acc_hbm, shacc, pbuf, idxv):
    cid = lax.axis_index("c")
    sid = lax.axis_index("s")
    nbase = sid * _NPS
    pltpu.sync_copy(zero_hbm.at[pl.ds(nbase, _NPS)], shacc.at[pl.ds(nbase, _NPS)])
    plsc.subcore_barrier()
    wbase = (cid * 16 + sid) * _EPW

    def body(c, carry):
        base = pl.multiple_of(wbase + c * _CES, 8)
        pltpu.sync_copy(dst_hbm.at[pl.ds(base, _CES)], idxv)
        pltpu.sync_copy(pay_hbm.at[pl.ds(base, _CES)], pbuf)
        pltpu.sync_copy(pbuf, shacc.at[idxv], add=True)
        return carry

    lax.fori_loop(0, _NCHS, body, 0)
    plsc.subcore_barrier()
    pltpu.sync_copy(shacc.at[pl.ds(nbase, _NPS)], acc_hbm.at[cid, pl.ds(nbase, _NPS)])


def _edge_body(gs_ref, gd_ref, e_ref,
               c_ref, r_ref, b1_ref, w2_ref, b2_ref, wx_ref, bx_ref,
               pay_ref, en_ref):
    gs = gs_ref[...]
    gd = gd_ref[...]
    himask = _u32(0xFFFF0000)
    sh = _u32(16)
    ps_w = gs[:, 0:64]
    pd_w = gd[:, 0:64]
    lo = (lax.bitcast_convert_type(ps_w << sh, _f32)
          + lax.bitcast_convert_type(pd_w << sh, _f32))
    hi = (lax.bitcast_convert_type(ps_w & himask, _f32)
          + lax.bitcast_convert_type(pd_w & himask, _f32))
    z = jnp.concatenate([lo, hi], axis=1)
    xs = lax.bitcast_convert_type(gs[:, 64:67], _f32)
    xd = lax.bitcast_convert_type(gd[:, 64:67], _f32)
    rel = xs - xd
    d2 = jnp.sum(rel * rel, axis=1, keepdims=True)
    z = z + jnp.dot(e_ref[...], c_ref[...], preferred_element_type=_f32)
    z = z + d2 * r_ref[...] + b1_ref[...]
    m = _silu(z)
    en = jnp.dot(m, w2_ref[...], preferred_element_type=_f32) + b2_ref[...]
    en = _silu(en)
    w = jnp.tanh(jnp.dot(en, wx_ref[...], preferred_element_type=_f32)
                 + bx_ref[...])[:, 0:1]
    relw = rel * w
    ones = jnp.ones((_TE, 1), _f32)
    zeros = jnp.zeros((_TE, 108), _f32)
    pay_ref[...] = jnp.concatenate([en, relw, ones, zeros], axis=1)
    en_ref[...] = en


def _tc_edge(Gs, Gd, e, C, r, b1, W2, b2, Wx8, bx8):
    zz = lambda i: (0, 0)
    return pl.pallas_call(
        _edge_body,
        grid=(_E // _TE,),
        in_specs=[
            pl.BlockSpec((_TE, _F), lambda i: (i, 0)),
            pl.BlockSpec((_TE, _F), lambda i: (i, 0)),
            pl.BlockSpec((_TE, 16), lambda i: (i, 0)),
            pl.BlockSpec((_De, _H), zz),
            pl.BlockSpec((1, _H), zz),
            pl.BlockSpec((1, _H), zz),
            pl.BlockSpec((_H, _De), zz),
            pl.BlockSpec((1, _De), zz),
            pl.BlockSpec((_De, 8), zz),
            pl.BlockSpec((1, 8), zz),
        ],
        out_specs=[
            pl.BlockSpec((_TE, _F), lambda i: (i, 0)),
            pl.BlockSpec((_TE, 16), lambda i: (i, 0)),
        ],
        out_shape=[
            jax.ShapeDtypeStruct((_E, _F), _f32),
            jax.ShapeDtypeStruct((_E, 16), _f32),
        ],
    )(Gs, Gd, e, C, r, b1, W2, b2, Wx8, bx8)


def _node_body(h_ref, xp_ref, acc_ref, w1h_ref, w1a_ref, b1_ref, w2_ref,
               b2_ref, hn_ref, xn_ref):
    acc = acc_ref[0] + acc_ref[1]
    deg = jnp.maximum(acc[:, 19:20], 1.0)
    agg = acc[:, 0:16] / deg
    dx = acc[:, 16:19] / deg
    xn_ref[...] = xp_ref[...] + jnp.concatenate(
        [dx, jnp.zeros((_TN, 125), _f32)], axis=1)
    h = h_ref[...]
    pre = (jnp.dot(h, w1h_ref[...], preferred_element_type=_f32)
           + jnp.dot(agg, w1a_ref[...], preferred_element_type=_f32)
           + b1_ref[...])
    hn_ref[...] = h + jnp.dot(_silu(pre), w2_ref[...],
                              preferred_element_type=_f32) + b2_ref[...]


def _tc_node(h, xp, acc2, W1h, W1a, b1, W2, b2):
    zz = lambda i: (0, 0)
    return pl.pallas_call(
        _node_body,
        grid=(_N // _TN,),
        in_specs=[
            pl.BlockSpec((_TN, _F), lambda i: (i, 0)),
            pl.BlockSpec((_TN, _F), lambda i: (i, 0)),
            pl.BlockSpec((2, _TN, _F), lambda i: (0, i, 0)),
            pl.BlockSpec((_F, _H), zz),
            pl.BlockSpec((16, _H), zz),
            pl.BlockSpec((1, _H), zz),
            pl.BlockSpec((_H, _F), zz),
            pl.BlockSpec((1, _F), zz),
        ],
        out_specs=[
            pl.BlockSpec((_TN, _F), lambda i: (i, 0)),
            pl.BlockSpec((_TN, _F), lambda i: (i, 0)),
        ],
        out_shape=[
            jax.ShapeDtypeStruct((_N, _F), _f32),
            jax.ShapeDtypeStruct((_N, _F), _f32),
        ],
    )(h, xp, acc2, W1h, W1a, b1, W2, b2)


def _head_body(hm_ref, hd_ref, hq_ref, ho_ref, nf_ref, gid_ref,
               wm_ref, bm_ref, wd_ref, bd_ref, wq_ref, bq_ref, wo_ref, bo_ref,
               pm_ref, pd_ref, pq_ref, po_ref, gs_ref):
    pm = jnp.dot(hm_ref[...], wm_ref[...], preferred_element_type=_f32) + bm_ref[...]
    mask = nf_ref[:, 0:1] == 1.0
    pm = jnp.where(mask, jnp.abs(pm), pm)
    pm_ref[...] = pm

    pd_ref[...] = jnp.dot(hd_ref[...], wd_ref[...],
                          preferred_element_type=_f32) + bd_ref[...]

    pq = jnp.dot(hq_ref[...], wq_ref[...], preferred_element_type=_f32) + bq_ref[...]
    i8 = lax.broadcasted_iota(jnp.int32, (1, 8), 1)
    mq = ((i8 == 0) | (i8 == 3) | (i8 == 5)).astype(_f32)
    mt = (pq[:, 0:1] + pq[:, 3:4] + pq[:, 5:6]) / 3.0
    pq_ref[...] = pq - mt * mq

    po = jnp.dot(ho_ref[...], wo_ref[...], preferred_element_type=_f32) + bo_ref[...]
    i16 = lax.broadcasted_iota(jnp.int32, (1, 16), 1)
    m0 = ((i16 == 0) | (i16 == 3) | (i16 == 5)).astype(_f32)
    m1 = ((i16 == 6) | (i16 == 1) | (i16 == 8)).astype(_f32)
    m2 = ((i16 == 9) | (i16 == 2) | (i16 == 7)).astype(_f32)
    t0 = (po[:, 0:1] + po[:, 3:4] + po[:, 5:6]) / 3.0
    t1 = (po[:, 6:7] + po[:, 1:2] + po[:, 8:9]) / 3.0
    t2 = (po[:, 9:10] + po[:, 2:3] + po[:, 7:8]) / 3.0
    po_ref[...] = po - t0 * m0 - t1 * m1 - t2 * m2

    gid = gid_ref[...]
    onehot = (lax.broadcasted_iota(jnp.int32, (_TN, _GP), 1) == gid).astype(_f32)
    ssum = lax.dot_general(onehot, pm[:, 0:1], (((0,), (0,)), ((), ())),
                           preferred_element_type=_f32)
    cnt = lax.dot_general(onehot, jnp.ones((_TN, 1), _f32),
                          (((0,), (0,)), ((), ())), preferred_element_type=_f32)
    locg = jnp.concatenate([ssum, cnt, jnp.zeros((_GP, 6), _f32)], axis=1)

    @pl.when(pl.program_id(0) == 0)
    def _():
        gs_ref[...] = locg

    @pl.when(pl.program_id(0) > 0)
    def _():
        gs_ref[...] = gs_ref[...] + locg


def _tc_head(hm, hd, hq, ho, nf, gid2, Wm8, bm8, Wd8, bd8, Wq8, bq8, Wo16, bo16):
    zz = lambda i: (0, 0)
    ii = lambda i: (i, 0)
    return pl.pallas_call(
        _head_body,
        grid=(_N // _TN,),
        in_specs=[
            pl.BlockSpec((_TN, _F), ii),
            pl.BlockSpec((_TN, _F), ii),
            pl.BlockSpec((_TN, _F), ii),
            pl.BlockSpec((_TN, _F), ii),
            pl.BlockSpec((_TN, _F), ii),
            pl.BlockSpec((_TN, 1), ii),
            pl.BlockSpec((_F, 8), zz),
            pl.BlockSpec((1, 8), zz),
            pl.BlockSpec((_F, 8), zz),
            pl.BlockSpec((1, 8), zz),
            pl.BlockSpec((_F, 8), zz),
            pl.BlockSpec((1, 8), zz),
            pl.BlockSpec((_F, 16), zz),
            pl.BlockSpec((1, 16), zz),
        ],
        out_specs=[
            pl.BlockSpec((_TN, 8), ii),
            pl.BlockSpec((_TN, 8), ii),
            pl.BlockSpec((_TN, 8), ii),
            pl.BlockSpec((_TN, 16), ii),
            pl.BlockSpec((_GP, 8), zz),
        ],
        out_shape=[
            jax.ShapeDtypeStruct((_N, 8), _f32),
            jax.ShapeDtypeStruct((_N, 8), _f32),
            jax.ShapeDtypeStruct((_N, 8), _f32),
            jax.ShapeDtypeStruct((_N, 16), _f32),
            jax.ShapeDtypeStruct((_GP, 8), _f32),
        ],
    )(hm, hd, hq, ho, nf, gid2, Wm8, bm8, Wd8, bd8, Wq8, bq8, Wo16, bo16)


def _fix_body(pm_ref, gid_ref, gs_ref, out_ref):
    sums = gs_ref[:, 0:1]
    cnts = jnp.maximum(gs_ref[:, 1:2], 1.0)
    fv = jnp.where(jnp.abs(sums) < 0.01, 0.0, sums / cnts)
    gid = gid_ref[...]
    onehot = (lax.broadcasted_iota(jnp.int32, (_TN, _GP), 1) == gid).astype(_f32)
    fvg = jnp.dot(onehot, fv, preferred_element_type=_f32)
    out_ref[...] = pm_ref[...] - fvg


def _tc_fix(pm_raw, gid2, gsums):
    ii = lambda i: (i, 0)
    return pl.pallas_call(
        _fix_body,
        grid=(_N // _TN,),
        in_specs=[
            pl.BlockSpec((_TN, 8), ii),
            pl.BlockSpec((_TN, 1), ii),
            pl.BlockSpec((_GP, 8), lambda i: (0, 0)),
        ],
        out_specs=pl.BlockSpec((_TN, 8), ii),
        out_shape=jax.ShapeDtypeStruct((_N, 8), _f32),
    )(pm_raw, gid2, gsums)


def kernel(nfeats, coordinates, efeats, edge_index, node_graph_ids,
           We1, be1, We2, be2, Wx, bx, Wh1, bh1, Wh2, bh2,
           Wm, bm, Wd, bd, Wq, bq, Wo, bo):
    src = edge_index[0]
    dst = edge_index[1]

    sigma = np.concatenate([np.arange(0, _H, 2), np.arange(1, _H, 2)])

    A_all = We1[:, :_F, :]
    B_all = We1[:, _F:2 * _F, :]
    C_all = We1[:, 2 * _F:2 * _F + _De, :][:, :, sigma]
    r_all = We1[:, 2 * _F + _De, :][:, sigma].reshape(_NCONV, 1, _H)
    b1_all = be1[:, sigma].reshape(_NCONV, 1, _H)
    W2_all = We2[:, sigma, :]
    b2_all = be2.reshape(_NCONV, 1, _De)
    Wx8 = jnp.concatenate([Wx, jnp.zeros((_NCONV, _De, 7), _f32)], axis=2)
    bx8 = jnp.concatenate([bx.reshape(_NCONV, 1, 1),
                           jnp.zeros((_NCONV, 1, 7), _f32)], axis=2)
    W1h_all = Wh1[:, :_F, :]
    W1a_all = Wh1[:, _F:, :]
    bh1r = bh1.reshape(_NCONV, 1, _H)
    bh2r = bh2.reshape(_NCONV, 1, _F)

    xp0 = jnp.concatenate([coordinates, jnp.zeros((_N, 125), _f32)], axis=1)
    zeros_acc = jnp.zeros((_NP, _F), _f32)

    def layer(i, h, xp, e):
        Ps, Pd = _tc_prep(h, A_all[i], B_all[i])
        Ts = _pack_table(Ps, xp)
        Td = _pack_table(Pd, xp)
        Gs, Gd = _sc_gather(Ts, Td, src, dst)
        pay, en = _tc_edge(Gs, Gd, e, C_all[i], r_all[i], b1_all[i],
                           W2_all[i], b2_all[i], Wx8[i], bx8[i])
        acc2 = _sc_scatter(pay, dst, zeros_acc)
        h2, xp2 = _tc_node(h, xp, acc2, W1h_all[i], W1a_all[i], bh1r[i],
                           Wh2[i], bh2r[i])
        return h2, xp2, en

    def branch(b):
        h, xp, e = nfeats, xp0, efeats
        for l in range(5):
            h, xp, e = layer(b * 5 + l, h, xp, e)
        return h

    h_mon = branch(0)
    h_dip = branch(1)
    h_quad = branch(2)
    h_oct = branch(3)

    gid2 = node_graph_ids.reshape(_N, 1).astype(jnp.int32)
    Wm8 = jnp.concatenate([Wm, jnp.zeros((_F, 7), _f32)], axis=1)
    bm8 = jnp.concatenate([bm.reshape(1, 1), jnp.zeros((1, 7), _f32)], axis=1)
    Wd8 = jnp.concatenate([Wd, jnp.zeros((_F, 5), _f32)], axis=1)
    bd8 = jnp.concatenate([bd.reshape(1, 3), jnp.zeros((1, 5), _f32)], axis=1)
    Wq8 = jnp.concatenate([Wq, jnp.zeros((_F, 2), _f32)], axis=1)
    bq8 = jnp.concatenate([bq.reshape(1, 6), jnp.zeros((1, 2), _f32)], axis=1)
    Wo16 = jnp.concatenate([Wo, jnp.zeros((_F, 6), _f32)], axis=1)
    bo16 = jnp.concatenate([bo.reshape(1, 10), jnp.zeros((1, 6), _f32)], axis=1)

    pm_raw, pd, pq, po, gsums = _tc_head(
        h_mon, h_dip, h_quad, h_oct, nfeats, gid2,
        Wm8, bm8, Wd8, bd8, Wq8, bq8, Wo16, bo16)
    pm = _tc_fix(pm_raw, gid2, gsums)

    return jnp.concatenate(
        [pm[:, :1], pd[:, :3], pq[:, :6], po[:, :10]], axis=1)
